# baseline (speedup 1.0000x reference)
import jax
import jax.numpy as jnp
from jax.experimental import pallas as pl


_ROWS = 16384
_COLS = 2048
_BLK = 1024
_NB = _ROWS // _BLK


def _concat_kernel(left_ref, right_ref, out_ref):
    j = pl.program_id(0)

    @pl.when(j == 0)
    def _():
        out_ref[:] = left_ref[:]

    @pl.when(j == 1)
    def _():
        out_ref[:] = right_ref[:]


def kernel(left, right):
    return pl.pallas_call(
        _concat_kernel,
        grid=(2, _NB),
        in_specs=[
            pl.BlockSpec((_BLK, _COLS), lambda j, i: ((1 - j) * i + j * (_NB - 1), 0)),
            pl.BlockSpec((_BLK, _COLS), lambda j, i: (j * i, 0)),
        ],
        out_specs=pl.BlockSpec((_BLK, _COLS), lambda j, i: (i, j)),
        out_shape=jax.ShapeDtypeStruct((_ROWS, 2 * _COLS), jnp.float32),
    )(left, right)


# speedup vs baseline: 1.0013x; 1.0013x over previous
import jax
import jax.numpy as jnp
from jax.experimental import pallas as pl
from jax.experimental.pallas import tpu as pltpu


_ROWS = 16384
_COLS = 2048
_BLK = 512


def _concat_kernel(left_ref, right_ref, out_ref):
    out_ref[:, :_COLS] = left_ref[:]
    out_ref[:, _COLS:] = right_ref[:]


def kernel(left, right):
    n_blk = _ROWS // _BLK
    return pl.pallas_call(
        _concat_kernel,
        grid=(n_blk,),
        in_specs=[
            pl.BlockSpec((_BLK, _COLS), lambda i: (i, 0)),
            pl.BlockSpec((_BLK, _COLS), lambda i: (i, 0)),
        ],
        out_specs=pl.BlockSpec((_BLK, 2 * _COLS), lambda i: (i, 0)),
        out_shape=jax.ShapeDtypeStruct((_ROWS, 2 * _COLS), jnp.float32),
        compiler_params=pltpu.CompilerParams(
            dimension_semantics=("arbitrary",),
            disable_bounds_checks=True,
            disable_semaphore_checks=True,
        ),
    )(left, right)
